# Initial kernel scaffold; baseline (speedup 1.0000x reference)
import jax
import jax.numpy as jnp
from jax.experimental import pallas as pl


def kernel(x, codebooks):
    raise NotImplementedError("write your pallas kernel here")


# speedup vs baseline: 1.5557x; 1.5557x over previous
import functools

import jax
import jax.numpy as jnp
from jax.experimental import pallas as pl
import jax.experimental.pallas.tpu as pltpu

COMMIT_W = 0.1


def _rvq_kernel(x_ref, cb_ref, xq_ref, codes_ref, loss_ref, subq_ref,
                dist_ref, res_ref, *, n_tokens):
    i = pl.program_id(0)
    q = pl.program_id(1)
    K = cb_ref.shape[1]

    @pl.when(q == 0)
    def _():
        res_ref[...] = x_ref[...]

    residual = res_ref[...]
    cb = cb_ref[q]

    r2 = jnp.sum(residual * residual, axis=1, keepdims=True)
    cb2 = jnp.sum(cb * cb, axis=1)
    xe = jax.lax.dot_general(residual, cb, (((1,), (1,)), ((), ())),
                             preferred_element_type=jnp.float32)
    d = (r2 - 2.0 * xe) + cb2[None, :]

    @pl.when(q == 0)
    def _():
        dist_ref[...] = d

    min_d = jnp.min(d, axis=1, keepdims=True)
    lane = jax.lax.broadcasted_iota(jnp.int32, d.shape, 1)
    idx = jnp.min(jnp.where(d == min_d, lane, K), axis=1)
    codes_ref[...] = idx.reshape(codes_ref.shape)

    onehot = (lane == idx[:, None]).astype(jnp.bfloat16)
    cb_hi = cb.astype(jnp.bfloat16)
    r1 = cb - cb_hi.astype(jnp.float32)
    cb_mid = r1.astype(jnp.bfloat16)
    cb_lo = (r1 - cb_mid.astype(jnp.float32)).astype(jnp.bfloat16)
    dn = (((1,), (0,)), ((), ()))
    q_hi = jax.lax.dot_general(onehot, cb_hi, dn,
                               preferred_element_type=jnp.float32)
    q_mid = jax.lax.dot_general(onehot, cb_mid, dn,
                                preferred_element_type=jnp.float32)
    q_lo = jax.lax.dot_general(onehot, cb_lo, dn,
                               preferred_element_type=jnp.float32)
    quant = (q_hi + q_mid) + q_lo
    subq_ref[0] = quant

    quant_st = residual + (quant - residual)

    @pl.when(q == 0)
    def _():
        xq_ref[...] = quant_st

    @pl.when(q > 0)
    def _():
        xq_ref[...] += quant_st

    partial = jnp.sum(min_d).reshape(1, 1)

    @pl.when((i == 0) & (q == 0))
    def _():
        loss_ref[...] = partial

    @pl.when((i > 0) | (q > 0))
    def _():
        loss_ref[...] = loss_ref[...] + partial

    res_ref[...] = residual - quant


@functools.partial(jax.jit, static_argnames=("block_n",))
def _rvq_forward(x, codebooks, block_n=None):
    B, T, D = x.shape
    NQ, K, _ = codebooks.shape
    N = B * T
    if block_n is None:
        block_n = max(bn for bn in range(8, min(N, 1024) + 1, 8) if N % bn == 0)
    BN = block_n
    nblk = N // BN
    flat = x.reshape(N, D)

    grid = (nblk, NQ)
    xq, codes, loss, subq, dist = pl.pallas_call(
        functools.partial(_rvq_kernel, n_tokens=N),
        grid=grid,
        in_specs=[
            pl.BlockSpec((BN, D), lambda i, q: (i, 0)),
            pl.BlockSpec((NQ, K, D), lambda i, q: (0, 0, 0)),
        ],
        out_specs=[
            pl.BlockSpec((BN, D), lambda i, q: (i, 0)),
            pl.BlockSpec((1, 1, 1, BN), lambda i, q: (q, i, 0, 0)),
            pl.BlockSpec((1, 1), lambda i, q: (0, 0)),
            pl.BlockSpec((1, BN, D), lambda i, q: (q, i, 0)),
            pl.BlockSpec((BN, K), lambda i, q: (i, 0)),
        ],
        out_shape=[
            jax.ShapeDtypeStruct((N, D), jnp.float32),
            jax.ShapeDtypeStruct((NQ, nblk, 1, BN), jnp.int32),
            jax.ShapeDtypeStruct((1, 1), jnp.float32),
            jax.ShapeDtypeStruct((NQ, N, D), jnp.float32),
            jax.ShapeDtypeStruct((N, K), jnp.float32),
        ],
        scratch_shapes=[pltpu.VMEM((BN, D), jnp.float32)],
    )(flat, codebooks)

    commit_loss = (loss[0, 0] * (COMMIT_W / (N * D))).astype(jnp.float32)
    return (xq.reshape(B, T, D),
            codes.reshape(NQ, B, T),
            commit_loss,
            subq.reshape(NQ, B, T, D),
            dist.reshape(B, T, K))


def kernel(x, codebooks):
    return _rvq_forward(x, codebooks)


# speedup vs baseline: 1.5557x; 1.0000x over previous
import functools

import jax
import jax.numpy as jnp
from jax.experimental import pallas as pl
import jax.experimental.pallas.tpu as pltpu

COMMIT_W = 0.1


def _rvq_kernel(x_ref, cbh_ref, cbm_ref, cbl_ref, cb2_ref,
                xq_ref, codes_ref, loss_ref, subq_ref, dist_ref, res_ref):
    i = pl.program_id(0)
    q = pl.program_id(1)
    K = cbh_ref.shape[1]

    @pl.when(q == 0)
    def _():
        res_ref[...] = x_ref[...]

    residual = res_ref[...]
    cb_h32 = cbh_ref[q].astype(jnp.float32)
    cb_m32 = cbm_ref[q].astype(jnp.float32)
    cb_l32 = cbl_ref[q].astype(jnp.float32)
    cb = (cb_h32 + cb_m32) + cb_l32

    r2 = jnp.sum(residual * residual, axis=1, keepdims=True)
    cb2 = cb2_ref[q]
    xe = jax.lax.dot_general(residual, cb, (((1,), (1,)), ((), ())),
                             preferred_element_type=jnp.float32)
    d = (r2 - 2.0 * xe) + cb2

    @pl.when(q == 0)
    def _():
        dist_ref[...] = d

    min_d = jnp.min(d, axis=1, keepdims=True)
    lane = jax.lax.broadcasted_iota(jnp.int32, d.shape, 1)
    idx = jnp.min(jnp.where(d == min_d, lane, K), axis=1)
    codes_ref[...] = idx.reshape(codes_ref.shape)

    onehot = (lane == idx[:, None]).astype(jnp.bfloat16)
    dn = (((1,), (0,)), ((), ()))
    q_hi = jax.lax.dot_general(onehot, cbh_ref[q], dn,
                               preferred_element_type=jnp.float32)
    q_mid = jax.lax.dot_general(onehot, cbm_ref[q], dn,
                                preferred_element_type=jnp.float32)
    q_lo = jax.lax.dot_general(onehot, cbl_ref[q], dn,
                               preferred_element_type=jnp.float32)
    quant = (q_hi + q_mid) + q_lo
    subq_ref[0] = quant

    quant_st = residual + (quant - residual)

    @pl.when(q == 0)
    def _():
        xq_ref[...] = quant_st

    @pl.when(q > 0)
    def _():
        xq_ref[...] += quant_st

    partial = jnp.sum(min_d).reshape(1, 1)

    @pl.when((i == 0) & (q == 0))
    def _():
        loss_ref[...] = partial

    @pl.when((i > 0) | (q > 0))
    def _():
        loss_ref[...] = loss_ref[...] + partial

    res_ref[...] = residual - quant


@functools.partial(jax.jit, static_argnames=("block_n",))
def _rvq_forward(x, codebooks, block_n=None):
    B, T, D = x.shape
    NQ, K, _ = codebooks.shape
    N = B * T
    if block_n is None:
        block_n = max(bn for bn in range(8, min(N, 1024) + 1, 8) if N % bn == 0)
    BN = block_n
    nblk = N // BN
    flat = x.reshape(N, D)

    cb_hi = codebooks.astype(jnp.bfloat16)
    r1 = codebooks - cb_hi.astype(jnp.float32)
    cb_mid = r1.astype(jnp.bfloat16)
    cb_lo = (r1 - cb_mid.astype(jnp.float32)).astype(jnp.bfloat16)
    cb2 = jnp.sum(codebooks ** 2, axis=2)[:, None, :]

    grid = (nblk, NQ)
    xq, codes, loss, subq, dist = pl.pallas_call(
        _rvq_kernel,
        grid=grid,
        in_specs=[
            pl.BlockSpec((BN, D), lambda i, q: (i, 0)),
            pl.BlockSpec((NQ, K, D), lambda i, q: (0, 0, 0)),
            pl.BlockSpec((NQ, K, D), lambda i, q: (0, 0, 0)),
            pl.BlockSpec((NQ, K, D), lambda i, q: (0, 0, 0)),
            pl.BlockSpec((NQ, 1, K), lambda i, q: (0, 0, 0)),
        ],
        out_specs=[
            pl.BlockSpec((BN, D), lambda i, q: (i, 0)),
            pl.BlockSpec((1, 1, 1, BN), lambda i, q: (q, i, 0, 0)),
            pl.BlockSpec((1, 1), lambda i, q: (0, 0)),
            pl.BlockSpec((1, BN, D), lambda i, q: (q, i, 0)),
            pl.BlockSpec((BN, K), lambda i, q: (i, 0)),
        ],
        out_shape=[
            jax.ShapeDtypeStruct((N, D), jnp.float32),
            jax.ShapeDtypeStruct((NQ, nblk, 1, BN), jnp.int32),
            jax.ShapeDtypeStruct((1, 1), jnp.float32),
            jax.ShapeDtypeStruct((NQ, N, D), jnp.float32),
            jax.ShapeDtypeStruct((N, K), jnp.float32),
        ],
        scratch_shapes=[pltpu.VMEM((BN, D), jnp.float32)],
    )(flat, cb_hi, cb_mid, cb_lo, cb2)

    commit_loss = (loss[0, 0] * (COMMIT_W / (N * D))).astype(jnp.float32)
    return (xq.reshape(B, T, D),
            codes.reshape(NQ, B, T),
            commit_loss,
            subq.reshape(NQ, B, T, D),
            dist.reshape(B, T, K))


def kernel(x, codebooks):
    return _rvq_forward(x, codebooks)


# speedup vs baseline: 1.8380x; 1.1815x over previous
import functools

import jax
import jax.numpy as jnp
from jax.experimental import pallas as pl
import jax.experimental.pallas.tpu as pltpu

COMMIT_W = 0.1


def _rvq_kernel(x_ref, cb_ref, xq_ref, codes_ref, loss_ref, subq_ref,
                dist_ref, res_ref):
    i = pl.program_id(0)
    q = pl.program_id(1)
    K = cb_ref.shape[1]

    @pl.when(q == 0)
    def _():
        res_ref[...] = x_ref[0]

    residual = res_ref[...]
    cb = cb_ref[0]

    r2 = jnp.sum(residual * residual, axis=1, keepdims=True)
    cb2 = jnp.sum(cb * cb, axis=1)
    xe = jax.lax.dot_general(residual, cb, (((1,), (1,)), ((), ())),
                             preferred_element_type=jnp.float32)
    d = (r2 - 2.0 * xe) + cb2[None, :]

    @pl.when(q == 0)
    def _():
        dist_ref[0] = d

    min_d = jnp.min(d, axis=1, keepdims=True)
    lane = jax.lax.broadcasted_iota(jnp.int32, d.shape, 1)
    idx = jnp.min(jnp.where(d == min_d, lane, K), axis=1)
    codes_ref[...] = idx.reshape(codes_ref.shape)

    onehot = (lane == idx[:, None]).astype(jnp.bfloat16)
    cb_hi = cb.astype(jnp.bfloat16)
    r1 = cb - cb_hi.astype(jnp.float32)
    cb_mid = r1.astype(jnp.bfloat16)
    cb_lo = (r1 - cb_mid.astype(jnp.float32)).astype(jnp.bfloat16)
    dn = (((1,), (0,)), ((), ()))
    q_hi = jax.lax.dot_general(onehot, cb_hi, dn,
                               preferred_element_type=jnp.float32)
    q_mid = jax.lax.dot_general(onehot, cb_mid, dn,
                                preferred_element_type=jnp.float32)
    q_lo = jax.lax.dot_general(onehot, cb_lo, dn,
                               preferred_element_type=jnp.float32)
    quant = (q_hi + q_mid) + q_lo
    subq_ref[0, 0] = quant

    quant_st = residual + (quant - residual)

    @pl.when(q == 0)
    def _():
        xq_ref[0] = quant_st

    @pl.when(q > 0)
    def _():
        xq_ref[0] += quant_st

    partial = jnp.sum(min_d).reshape(1, 1)

    @pl.when((i == 0) & (q == 0))
    def _():
        loss_ref[...] = partial

    @pl.when((i > 0) | (q > 0))
    def _():
        loss_ref[...] = loss_ref[...] + partial

    res_ref[...] = residual - quant


@functools.partial(jax.jit, static_argnames=("block_n",))
def _rvq_forward(x, codebooks, block_n=None):
    B, T, D = x.shape
    NQ, K, _ = codebooks.shape
    N = B * T
    BN = T if block_n is None else block_n
    nsub = T // BN
    nblk = B * nsub

    grid = (nblk, NQ)
    xq, codes, loss, subq, dist = pl.pallas_call(
        _rvq_kernel,
        grid=grid,
        in_specs=[
            pl.BlockSpec((1, BN, D), lambda i, q: (i // nsub, i % nsub, 0)),
            pl.BlockSpec((1, K, D), lambda i, q: (q, 0, 0)),
        ],
        out_specs=[
            pl.BlockSpec((1, BN, D), lambda i, q: (i // nsub, i % nsub, 0)),
            pl.BlockSpec((1, 1, 1, BN), lambda i, q: (q, i, 0, 0)),
            pl.BlockSpec((1, 1), lambda i, q: (0, 0)),
            pl.BlockSpec((1, 1, BN, D),
                         lambda i, q: (q, i // nsub, i % nsub, 0)),
            pl.BlockSpec((1, BN, K), lambda i, q: (i // nsub, i % nsub, 0)),
        ],
        out_shape=[
            jax.ShapeDtypeStruct((B, T, D), jnp.float32),
            jax.ShapeDtypeStruct((NQ, nblk, 1, BN), jnp.int32),
            jax.ShapeDtypeStruct((1, 1), jnp.float32),
            jax.ShapeDtypeStruct((NQ, B, T, D), jnp.float32),
            jax.ShapeDtypeStruct((B, T, K), jnp.float32),
        ],
        scratch_shapes=[pltpu.VMEM((BN, D), jnp.float32)],
    )(x, codebooks)

    commit_loss = (loss[0, 0] * (COMMIT_W / (N * D))).astype(jnp.float32)
    return (xq,
            codes.reshape(NQ, B, T),
            commit_loss,
            subq,
            dist)


def kernel(x, codebooks):
    return _rvq_forward(x, codebooks)
